# baseline (speedup 1.0000x reference)
import functools

import jax
import jax.numpy as jnp
from jax import lax
from jax.experimental import pallas as pl
from jax.experimental.pallas import tpu as pltpu
from jax.experimental.pallas import tpu_sc as plsc

B = 4096
L = 200
EMB = 32
N = B * L

_info = plsc.get_sparse_core_info()
NC = _info.num_cores
NS = _info.num_subcores
NW = NC * NS
BT = B // 128
NUNIT = L * BT
PER_W = NUNIT // NW
PER_W_IDX = PER_W * 128
NSLOT = 4

_mesh = plsc.VectorSubcoreMesh(core_axis_name="c", subcore_axis_name="s")


@functools.partial(
    pl.kernel,
    mesh=_mesh,
    compiler_params=pltpu.CompilerParams(
        use_tc_tiling_on_sc=False, needs_layout_passes=False),
    out_type=jax.ShapeDtypeStruct((L, 8, BT, 8, 128), jnp.float32),
    scratch_types=[
        pltpu.VMEM((PER_W_IDX,), jnp.int32),
        pltpu.VMEM((PER_W_IDX,), jnp.int32),
        [pltpu.VMEM((128, EMB), jnp.float32) for _ in range(NSLOT)],
        [pltpu.VMEM((128, EMB), jnp.float32) for _ in range(NSLOT)],
        [pltpu.VMEM((8, 8, 128), jnp.float32) for _ in range(NSLOT)],
        [pltpu.SemaphoreType.DMA for _ in range(NSLOT)],
        [pltpu.SemaphoreType.DMA for _ in range(NSLOT)],
    ],
)
def _sc_gather(idx0_hbm, idx1_hbm, t0_hbm, t1_hbm, out_hbm,
               idx0_v, idx1_v, g0, g1, nb, gsem, wsem):
    wid = lax.axis_index("s") * NC + lax.axis_index("c")
    ubase = wid * PER_W

    pltpu.sync_copy(idx0_hbm.at[pl.ds(ubase * 128, PER_W_IDX)], idx0_v)
    pltpu.sync_copy(idx1_hbm.at[pl.ds(ubase * 128, PER_W_IDX)], idx1_v)

    iota = lax.iota(jnp.int32, 16)
    rows = [iota + (16 * jj) for jj in range(8)]

    def fire_gathers(u, p):
        pltpu.async_copy(t0_hbm.at[idx0_v.at[pl.ds(u * 128, 128)]], g0[p], gsem[p])
        pltpu.async_copy(t1_hbm.at[idx1_v.at[pl.ds(u * 128, 128)]], g1[p], gsem[p])

    def wait_gathers(p):
        pltpu.make_async_copy(t0_hbm.at[pl.ds(0, 128)], g0[p], gsem[p]).wait()
        pltpu.make_async_copy(t1_hbm.at[pl.ds(0, 128)], g1[p], gsem[p]).wait()

    def fire_write(u, p):
        gu = ubase + u
        l = gu // BT
        bt = gu % BT
        pltpu.async_copy(nb[p], out_hbm.at[l, :, bt], wsem[p])

    def wait_write(p):
        pltpu.make_async_copy(nb[p], out_hbm.at[0, :, 0], wsem[p]).wait()

    def assemble(p):
        def body(e, carry):
            col = jnp.zeros((16,), jnp.int32) + e
            et0 = e >> 3
            ei = e & 7
            for jj in range(8):
                b0 = 16 * jj
                v0 = plsc.load_gather(g0[p], [rows[jj], col])
                nb[p][et0, ei, pl.ds(b0, 16)] = v0
                v1 = plsc.load_gather(g1[p], [rows[jj], col])
                nb[p][4 + et0, ei, pl.ds(b0, 16)] = v1
            return carry

        lax.fori_loop(0, EMB, body, 0)

    for p in range(NSLOT):
        fire_gathers(p, p)

    for i in range(NSLOT):
        wait_gathers(i)
        assemble(i)
        fire_write(i, i)
        fire_gathers(i + NSLOT, i)

    def round_body(r, carry):
        for i in range(NSLOT):
            u = r * NSLOT + i
            wait_gathers(i)
            wait_write(i)
            assemble(i)
            fire_write(u, i)
            fire_gathers(u + NSLOT, i)
        return carry

    lax.fori_loop(1, PER_W // NSLOT - 1, round_body, 0)

    for i in range(NSLOT):
        u = PER_W - NSLOT + i
        wait_gathers(i)
        wait_write(i)
        assemble(i)
        fire_write(u, i)
    for i in range(NSLOT):
        wait_write(i)


def _mask_body(h_ref, m_ref):
    m_ref[...] = h_ref[...] != 0


_mask_call = pl.pallas_call(
    _mask_body,
    out_shape=jax.ShapeDtypeStruct((B, L), jnp.bool_),
)


def kernel(histories, item_emb_0, item_emb_1):
    hist0 = histories[:, 0, :]
    idx0 = jnp.swapaxes(hist0, 0, 1).reshape(N)
    idx1 = jnp.swapaxes(histories[:, 1, :], 0, 1).reshape(N)
    out5 = _sc_gather(idx0, idx1, item_emb_0, item_emb_1)
    embs = out5.transpose(2, 4, 0, 1, 3).reshape(B, L, 2 * EMB)
    mask = _mask_call(hist0)
    return embs, mask


# speedup vs baseline: 1.4684x; 1.4684x over previous
import functools

import jax
import jax.numpy as jnp
from jax import lax
from jax.experimental import pallas as pl
from jax.experimental.pallas import tpu as pltpu
from jax.experimental.pallas import tpu_sc as plsc

B = 4096
L = 200
EMB = 32
N = B * L

_info = plsc.get_sparse_core_info()
NC = _info.num_cores
NS = _info.num_subcores
NW = NC * NS
BT = B // 128
NUNIT = L * BT
PER_W = NUNIT // NW
PER_W_IDX = PER_W * 128
NSLOT = 4

_mesh = plsc.VectorSubcoreMesh(core_axis_name="c", subcore_axis_name="s")


@functools.partial(
    pl.kernel,
    mesh=_mesh,
    compiler_params=pltpu.CompilerParams(
        use_tc_tiling_on_sc=False, needs_layout_passes=False),
    out_type=jax.ShapeDtypeStruct((L, 8, BT, 8, 128), jnp.float32),
    scratch_types=[
        pltpu.VMEM((PER_W_IDX,), jnp.int32),
        pltpu.VMEM((PER_W_IDX,), jnp.int32),
        [pltpu.VMEM((128, EMB), jnp.float32) for _ in range(NSLOT)],
        [pltpu.VMEM((128, EMB), jnp.float32) for _ in range(NSLOT)],
        [pltpu.VMEM((2 * EMB, 128), jnp.float32) for _ in range(NSLOT)],
        [pltpu.SemaphoreType.DMA for _ in range(NSLOT)],
        [pltpu.SemaphoreType.DMA for _ in range(NSLOT)],
    ],
)
def _sc_gather(idx0_hbm, idx1_hbm, t0_hbm, t1_hbm, out_hbm,
               idx0_v, idx1_v, g0, g1, nb, gsem, wsem):
    wid = lax.axis_index("s") * NC + lax.axis_index("c")
    ubase = wid * PER_W

    pltpu.sync_copy(idx0_hbm.at[pl.ds(ubase * 128, PER_W_IDX)], idx0_v)
    pltpu.sync_copy(idx1_hbm.at[pl.ds(ubase * 128, PER_W_IDX)], idx1_v)

    iota = lax.iota(jnp.int32, 16)
    diag = [(iota + d) & 15 for d in range(16)]

    def fire_gathers(u, p):
        pltpu.async_copy(t0_hbm.at[idx0_v.at[pl.ds(u * 128, 128)]], g0[p], gsem[p])
        pltpu.async_copy(t1_hbm.at[idx1_v.at[pl.ds(u * 128, 128)]], g1[p], gsem[p])

    def wait_gathers(p):
        pltpu.make_async_copy(t0_hbm.at[pl.ds(0, 128)], g0[p], gsem[p]).wait()
        pltpu.make_async_copy(t1_hbm.at[pl.ds(0, 128)], g1[p], gsem[p]).wait()

    def fire_write(u, p):
        gu = ubase + u
        l = gu // BT
        bt = gu % BT
        for et in range(8):
            pltpu.async_copy(nb[p].at[pl.ds(et * 8, 8)], out_hbm.at[l, et, bt],
                             wsem[p])

    def wait_write(p):
        for et in range(8):
            pltpu.make_async_copy(nb[p].at[pl.ds(0, 8)], out_hbm.at[0, 0, 0],
                                  wsem[p]).wait()

    def assemble(p):
        def sub(rb, carry):
            items = rb * 16 + iota
            for g, ebase in ((g0[p], 0), (g1[p], EMB)):
                for f0 in range(0, EMB, 16):
                    for d in range(16):
                        feats = diag[d] + f0
                        v = plsc.load_gather(g, [items, feats])
                        plsc.store_scatter(nb[p], [feats + ebase, items], v)
            return carry

        lax.fori_loop(0, 8, sub, 0)

    for p in range(NSLOT):
        fire_gathers(p, p)

    for i in range(NSLOT):
        wait_gathers(i)
        assemble(i)
        fire_write(i, i)
        fire_gathers(i + NSLOT, i)

    def round_body(r, carry):
        for i in range(NSLOT):
            u = r * NSLOT + i
            wait_gathers(i)
            wait_write(i)
            assemble(i)
            fire_write(u, i)
            fire_gathers(u + NSLOT, i)
        return carry

    lax.fori_loop(1, PER_W // NSLOT - 1, round_body, 0)

    for i in range(NSLOT):
        u = PER_W - NSLOT + i
        wait_gathers(i)
        wait_write(i)
        assemble(i)
        fire_write(u, i)
    for i in range(NSLOT):
        wait_write(i)


def _mask_body(h_ref, m_ref):
    m_ref[...] = h_ref[...] != 0


_mask_call = pl.pallas_call(
    _mask_body,
    out_shape=jax.ShapeDtypeStruct((B, L), jnp.bool_),
)


def kernel(histories, item_emb_0, item_emb_1):
    hist0 = histories[:, 0, :]
    idx0 = jnp.swapaxes(hist0, 0, 1).reshape(N)
    idx1 = jnp.swapaxes(histories[:, 1, :], 0, 1).reshape(N)
    out5 = _sc_gather(idx0, idx1, item_emb_0, item_emb_1)
    embs = out5.transpose(2, 4, 0, 1, 3).reshape(B, L, 2 * EMB)
    mask = _mask_call(hist0)
    return embs, mask


# speedup vs baseline: 1.8033x; 1.2281x over previous
import functools

import jax
import jax.numpy as jnp
from jax import lax
from jax.experimental import pallas as pl
from jax.experimental.pallas import tpu as pltpu
from jax.experimental.pallas import tpu_sc as plsc

B = 4096
L = 200
EMB = 32
N = B * L

_info = plsc.get_sparse_core_info()
NC = _info.num_cores
NS = _info.num_subcores
NW = NC * NS
BT = B // 128
NUNIT = L * BT
PER_W = NUNIT // NW
PER_W_IDX = PER_W * 128
NSLOT = 4

_mesh = plsc.VectorSubcoreMesh(core_axis_name="c", subcore_axis_name="s")


@functools.partial(
    pl.kernel,
    mesh=_mesh,
    compiler_params=pltpu.CompilerParams(
        use_tc_tiling_on_sc=False, needs_layout_passes=False),
    out_type=jax.ShapeDtypeStruct((L, 8, BT, 8, 128), jnp.float32),
    scratch_types=[
        pltpu.VMEM((PER_W_IDX,), jnp.int32),
        pltpu.VMEM((PER_W_IDX,), jnp.int32),
        [pltpu.VMEM((128, EMB), jnp.float32) for _ in range(NSLOT)],
        [pltpu.VMEM((128, EMB), jnp.float32) for _ in range(NSLOT)],
        [pltpu.VMEM((2 * EMB, 128), jnp.float32) for _ in range(NSLOT)],
        [pltpu.SemaphoreType.DMA for _ in range(NSLOT)],
        [pltpu.SemaphoreType.DMA for _ in range(NSLOT)],
    ],
)
def _sc_gather(idx0_hbm, idx1_hbm, t0_hbm, t1_hbm, out_hbm,
               idx0_v, idx1_v, g0, g1, nb, gsem, wsem):
    wid = lax.axis_index("s") * NC + lax.axis_index("c")
    ubase = wid * PER_W

    pltpu.sync_copy(idx0_hbm.at[pl.ds(ubase * 128, PER_W_IDX)], idx0_v)
    pltpu.sync_copy(idx1_hbm.at[pl.ds(ubase * 128, PER_W_IDX)], idx1_v)

    iota = lax.iota(jnp.int32, 16)
    diag = [(iota + d) & 15 for d in range(16)]

    def fire_gathers(u, p):
        pltpu.async_copy(t0_hbm.at[idx0_v.at[pl.ds(u * 128, 128)]], g0[p], gsem[p])
        pltpu.async_copy(t1_hbm.at[idx1_v.at[pl.ds(u * 128, 128)]], g1[p], gsem[p])

    def wait_gathers(p):
        pltpu.make_async_copy(t0_hbm.at[pl.ds(0, 128)], g0[p], gsem[p]).wait()
        pltpu.make_async_copy(t1_hbm.at[pl.ds(0, 128)], g1[p], gsem[p]).wait()

    def fire_write(u, p):
        gu = ubase + u
        l = gu // BT
        bt = gu % BT
        for et in range(8):
            pltpu.async_copy(nb[p].at[pl.ds(et * 8, 8)], out_hbm.at[l, et, bt],
                             wsem[p])

    def wait_write(p):
        for et in range(8):
            pltpu.make_async_copy(nb[p].at[pl.ds(0, 8)], out_hbm.at[0, 0, 0],
                                  wsem[p]).wait()

    def assemble(p):
        def sub(rb, carry):
            items = rb * 16 + iota
            for g, ebase in ((g0[p], 0), (g1[p], EMB)):
                for f0 in range(0, EMB, 16):
                    for d0 in range(0, 16, 4):
                        fv = [diag[d0 + k] + f0 for k in range(4)]
                        vs = [plsc.load_gather(g, [items, f]) for f in fv]
                        for f, v in zip(fv, vs):
                            plsc.store_scatter(nb[p], [f + ebase, items], v)
            return carry

        lax.fori_loop(0, 8, sub, 0)

    for p in range(NSLOT):
        fire_gathers(p, p)

    for i in range(NSLOT):
        wait_gathers(i)
        assemble(i)
        fire_write(i, i)
        fire_gathers(i + NSLOT, i)

    def round_body(r, carry):
        for i in range(NSLOT):
            u = r * NSLOT + i
            wait_gathers(i)
            wait_write(i)
            assemble(i)
            fire_write(u, i)
            fire_gathers(u + NSLOT, i)
        return carry

    lax.fori_loop(1, PER_W // NSLOT - 1, round_body, 0)

    for i in range(NSLOT):
        u = PER_W - NSLOT + i
        wait_gathers(i)
        wait_write(i)
        assemble(i)
        fire_write(u, i)
    for i in range(NSLOT):
        wait_write(i)


def _mask_body(h_ref, m_ref):
    m_ref[...] = h_ref[...] != 0


_mask_call = pl.pallas_call(
    _mask_body,
    out_shape=jax.ShapeDtypeStruct((B, L), jnp.bool_),
)


def kernel(histories, item_emb_0, item_emb_1):
    hist0 = histories[:, 0, :]
    idx0 = jnp.swapaxes(hist0, 0, 1).reshape(N)
    idx1 = jnp.swapaxes(histories[:, 1, :], 0, 1).reshape(N)
    out5 = _sc_gather(idx0, idx1, item_emb_0, item_emb_1)
    embs = out5.transpose(2, 4, 0, 1, 3).reshape(B, L, 2 * EMB)
    mask = _mask_call(hist0)
    return embs, mask
